# baseline (speedup 1.0000x reference)
import functools

import jax
import jax.numpy as jnp
from jax import lax
from jax.experimental import pallas as pl
from jax.experimental.pallas import tpu as pltpu
from jax.experimental.pallas import tpu_sc as plsc

B = 1024
IU = 128
DU = 32
DI = 32
HID = 256
CORPUS = 100000
NUM_ITEMS = 100

NC = 2
NS = 16
NW = NC * NS
ROWS_PER_W = B // NW

CB = 2048
CP = ((CORPUS + CB - 1) // CB) * CB
NEG = -3.0e38


def _sc_gather_body(table_hbm, idx_hbm, out_hbm, idx_v, rows_v, sem):
    wid = lax.axis_index("s") * NC + lax.axis_index("c")
    base = wid * ROWS_PER_W
    pltpu.sync_copy(idx_hbm.at[pl.ds(base, ROWS_PER_W)], idx_v)
    pltpu.async_copy(table_hbm.at[idx_v], rows_v, sem).wait()
    pltpu.sync_copy(rows_v, out_hbm.at[pl.ds(base, ROWS_PER_W)])


def _sc_gather(table, idx):
    mesh = plsc.VectorSubcoreMesh(core_axis_name="c", subcore_axis_name="s")
    return pl.kernel(
        _sc_gather_body,
        mesh=mesh,
        out_type=jax.ShapeDtypeStruct((B, DU), jnp.float32),
        scratch_types=[
            pltpu.VMEM((ROWS_PER_W,), jnp.int32),
            pltpu.VMEM((ROWS_PER_W, DU), jnp.float32),
            pltpu.SemaphoreType.DMA,
        ],
        compiler_params=pltpu.CompilerParams(use_tc_tiling_on_sc=False),
    )(table, idx)


def _tc_score_body(ue_ref, uf_ref, w1_ref, b1_ref, w2_ref, b2_ref,
                   wta_ref, wtb_ref, bt_ref, corpus_ref, scores_ref,
                   uemb_ref):
    i = pl.program_id(0)

    @pl.when(i == 0)
    def _():
        h = jnp.maximum(
            jnp.dot(uf_ref[...], w1_ref[...],
                    preferred_element_type=jnp.float32) + b1_ref[...], 0.0)
        ufe = jnp.dot(h, w2_ref[...],
                      preferred_element_type=jnp.float32) + b2_ref[...]
        uemb_ref[...] = (
            jnp.dot(ue_ref[...], wta_ref[...],
                    preferred_element_type=jnp.float32)
            + jnp.dot(ufe, wtb_ref[...], preferred_element_type=jnp.float32)
            + bt_ref[...])

    s = lax.dot_general(uemb_ref[...], corpus_ref[...],
                        (((1,), (1,)), ((), ())),
                        preferred_element_type=jnp.float32)
    col = i * CB + lax.broadcasted_iota(jnp.int32, (B, CB), 1)
    scores_ref[...] = jnp.where(col < CORPUS, s, NEG)


def _tc_scores(ue, uf, w1, b1, w2, b2, wta, wtb, bt, corpus_p):
    grid = CP // CB
    return pl.pallas_call(
        _tc_score_body,
        grid=(grid,),
        in_specs=[
            pl.BlockSpec((B, DU), lambda i: (0, 0)),
            pl.BlockSpec((B, IU), lambda i: (0, 0)),
            pl.BlockSpec((IU, HID), lambda i: (0, 0)),
            pl.BlockSpec((1, HID), lambda i: (0, 0)),
            pl.BlockSpec((HID, DU), lambda i: (0, 0)),
            pl.BlockSpec((1, DU), lambda i: (0, 0)),
            pl.BlockSpec((DU, DI), lambda i: (0, 0)),
            pl.BlockSpec((DU, DI), lambda i: (0, 0)),
            pl.BlockSpec((1, DI), lambda i: (0, 0)),
            pl.BlockSpec((CB, DI), lambda i: (i, 0)),
        ],
        out_specs=pl.BlockSpec((B, CB), lambda i: (0, i)),
        out_shape=jax.ShapeDtypeStruct((B, CP), jnp.float32),
        scratch_shapes=[pltpu.VMEM((B, DI), jnp.float32)],
    )(ue, uf, w1, b1, w2, b2, wta, wtb, bt, corpus_p)


def kernel(user_id, user_features, user_history, user_id_table,
           W1, b1, W2, b2, Wt, bt, corpus):
    del user_history
    ue = _sc_gather(user_id_table, user_id)
    corpus_p = jnp.pad(corpus, ((0, CP - CORPUS), (0, 0)))
    scores = _tc_scores(ue, user_features, W1, b1[None, :], W2, b2[None, :],
                        Wt[:DU], Wt[DU:], bt[None, :], corpus_p)
    _, top_items = lax.top_k(scores[:, :CORPUS], NUM_ITEMS)
    return top_items


# speedup vs baseline: 11.3870x; 11.3870x over previous
import functools

import jax
import jax.numpy as jnp
from jax import lax
from jax.experimental import pallas as pl
from jax.experimental.pallas import tpu as pltpu
from jax.experimental.pallas import tpu_sc as plsc

B = 1024
IU = 128
DU = 32
DI = 32
HID = 256
CORPUS = 100000
K = 100

NC = 2
NS = 16
NW = NC * NS
ROWS_PER_W = B // NW

CB = 2048
CP = ((CORPUS + CB - 1) // CB) * CB
SEGW = 128
NSEG = CP // SEGW
NSEGV = NSEG // 16
CAP = 512
NEG = -3.0e38
NINF = float("-inf")
IMAX = 2147483647
IMIN = -2147483648


def _sc_gather_body(table_hbm, idx_hbm, out_hbm, idx_v, rows_v, sem):
    wid = lax.axis_index("s") * NC + lax.axis_index("c")
    base = wid * ROWS_PER_W
    pltpu.sync_copy(idx_hbm.at[pl.ds(base, ROWS_PER_W)], idx_v)
    pltpu.async_copy(table_hbm.at[idx_v], rows_v, sem).wait()
    pltpu.sync_copy(rows_v, out_hbm.at[pl.ds(base, ROWS_PER_W)])


def _sc_gather(table, idx):
    mesh = plsc.VectorSubcoreMesh(core_axis_name="c", subcore_axis_name="s")
    return pl.kernel(
        _sc_gather_body,
        mesh=mesh,
        out_type=jax.ShapeDtypeStruct((B, DU), jnp.float32),
        scratch_types=[
            pltpu.VMEM((ROWS_PER_W,), jnp.int32),
            pltpu.VMEM((ROWS_PER_W, DU), jnp.float32),
            pltpu.SemaphoreType.DMA,
        ],
        compiler_params=pltpu.CompilerParams(use_tc_tiling_on_sc=False),
    )(table, idx)


def _tc_score_body(ue_ref, uf_ref, w1_ref, b1_ref, w2_ref, b2_ref,
                   wta_ref, wtb_ref, bt_ref, corpus_ref, scores_ref,
                   segmax_ref, uemb_ref):
    i = pl.program_id(0)

    @pl.when(i == 0)
    def _():
        h = jnp.maximum(
            jnp.dot(uf_ref[...], w1_ref[...],
                    preferred_element_type=jnp.float32) + b1_ref[...], 0.0)
        ufe = jnp.dot(h, w2_ref[...],
                      preferred_element_type=jnp.float32) + b2_ref[...]
        uemb_ref[...] = (
            jnp.dot(ue_ref[...], wta_ref[...],
                    preferred_element_type=jnp.float32)
            + jnp.dot(ufe, wtb_ref[...], preferred_element_type=jnp.float32)
            + bt_ref[...])

    s = lax.dot_general(uemb_ref[...], corpus_ref[...],
                        (((1,), (1,)), ((), ())),
                        preferred_element_type=jnp.float32)
    col = i * CB + lax.broadcasted_iota(jnp.int32, (B, CB), 1)
    s = jnp.where(col < CORPUS, s, NEG)
    scores_ref[...] = s
    segmax_ref[...] = jnp.max(s.reshape(B, CB // SEGW, SEGW), axis=2)[None]


def _tc_scores(ue, uf, w1, b1, w2, b2, wta, wtb, bt, corpus_p):
    grid = CP // CB
    return pl.pallas_call(
        _tc_score_body,
        grid=(grid,),
        in_specs=[
            pl.BlockSpec((B, DU), lambda i: (0, 0)),
            pl.BlockSpec((B, IU), lambda i: (0, 0)),
            pl.BlockSpec((IU, HID), lambda i: (0, 0)),
            pl.BlockSpec((1, HID), lambda i: (0, 0)),
            pl.BlockSpec((HID, DU), lambda i: (0, 0)),
            pl.BlockSpec((1, DU), lambda i: (0, 0)),
            pl.BlockSpec((DU, DI), lambda i: (0, 0)),
            pl.BlockSpec((DU, DI), lambda i: (0, 0)),
            pl.BlockSpec((1, DI), lambda i: (0, 0)),
            pl.BlockSpec((CB, DI), lambda i: (i, 0)),
        ],
        out_specs=[
            pl.BlockSpec((B, CB), lambda i: (0, i)),
            pl.BlockSpec((1, B, CB // SEGW), lambda i: (i, 0, 0)),
        ],
        out_shape=[
            jax.ShapeDtypeStruct((B, CP), jnp.float32),
            jax.ShapeDtypeStruct((grid, B, CB // SEGW), jnp.float32),
        ],
        scratch_shapes=[pltpu.VMEM((B, DI), jnp.float32)],
    )(ue, uf, w1, b1, w2, b2, wta, wtb, bt, corpus_p)


def _key16(v):
    bits = plsc.bitcast(v, jnp.int32)
    return jnp.where(bits >= 0, bits, bits ^ jnp.int32(0x7FFFFFFF))


def _inv_key16(kv):
    return plsc.bitcast(
        jnp.where(kv >= 0, kv, kv ^ jnp.int32(0x7FFFFFFF)), jnp.float32)


def _bsearch_kth(count_ge):
    cpos = count_ge(jnp.int32(0))
    t0 = jnp.where(cpos >= K, jnp.int32(0), jnp.int32(IMIN))

    def bit_body(b, t):
        bit = lax.shift_left(jnp.int32(1), jnp.int32(30) - b)
        tc = t + bit
        return jnp.where(count_ge(tc) >= K, tc, t)

    return lax.fori_loop(0, 31, bit_body, t0)


def _popcnt(m):
    return plsc.all_reduce_population_count(m)[0]


def _sc_topk_body(segs_hbm, segmax_hbm, out_hbm,
                  smax_v, skey_v, seglist_v, gbuf_v, cval_v, cidx_v,
                  outrow_v, nval_sm, sem):
    wid = lax.axis_index("s") * NC + lax.axis_index("c")
    iota = lax.iota(jnp.int32, 16)
    lane0 = iota == jnp.int32(0)
    ninf_v = jnp.full((16,), NINF, jnp.float32)

    def _clr_seg(i, _):
        seglist_v[pl.ds(16 * i, 16)] = jnp.zeros((16,), jnp.int32)
        return 0
    lax.fori_loop(0, seglist_v.shape[0] // 16, _clr_seg, 0)

    def do_row(r, _):
        row = wid * ROWS_PER_W + r
        rowseg = row * NSEG

        pltpu.sync_copy(segmax_hbm.at[row], smax_v)

        def kb(i, _):
            skey_v[pl.ds(16 * i, 16)] = _key16(smax_v[pl.ds(16 * i, 16)])
            return 0
        lax.fori_loop(0, NSEGV, kb, 0)

        def cnt_skey(t):
            tv = jnp.full((16,), t, jnp.int32)

            def cb(i, a):
                return a + _popcnt(skey_v[pl.ds(16 * i, 16)] >= tv)
            return lax.fori_loop(0, NSEGV, cb, jnp.int32(0))

        tseg = _bsearch_kth(cnt_skey)
        tseg_v = jnp.full((16,), tseg, jnp.int32)
        t0f = _inv_key16(tseg_v)

        def sb(i, p):
            m = skey_v[pl.ds(16 * i, 16)] >= tseg_v
            plsc.store_compressed(seglist_v.at[pl.ds(p, 16)],
                                  iota + (16 * i + rowseg), mask=m)
            return p + _popcnt(m)
        n_pass = lax.fori_loop(0, NSEGV, sb, jnp.int32(0))

        def gb(c, _):
            pltpu.async_copy(
                segs_hbm.at[seglist_v.at[pl.ds(128 * c, 128)]],
                gbuf_v.at[pl.ds(128 * c, 128)], sem).wait()
            return 0
        lax.fori_loop(0, (n_pass + 127) // 128, gb, 0)

        def _clr_cand(i, _):
            cval_v[pl.ds(16 * i, 16)] = ninf_v
            return 0

        def append_pass(thresh_v, strict, equal):
            def ab(s, ptr):
                gsid = plsc.load_gather(seglist_v,
                                        [jnp.full((16,), s, jnp.int32)])
                base = (gsid - rowseg) * SEGW

                def ub(u, pp):
                    v = gbuf_v[s, pl.ds(16 * u, 16)]
                    if equal:
                        m = _key16(v) == thresh_v
                    elif strict:
                        m = _key16(v) > thresh_v
                    else:
                        m = v >= _inv_key16(thresh_v)
                    pe = jnp.minimum(pp, jnp.int32(CAP))
                    plsc.store_compressed(cval_v.at[pl.ds(pe, 16)], v, mask=m)
                    plsc.store_compressed(cidx_v.at[pl.ds(pe, 16)],
                                          base + (iota + 16 * u), mask=m)
                    return pp + _popcnt(m)
                return lax.fori_loop(0, SEGW // 16, ub, ptr)
            return lax.fori_loop(0, n_pass, ab, jnp.int32(0))

        lax.fori_loop(0, cval_v.shape[0] // 16, _clr_cand, 0)
        ptr_fast = append_pass(tseg_v, False, False)
        nval_sm[0] = jnp.minimum(ptr_fast, jnp.int32(CAP))

        @pl.when(ptr_fast > jnp.int32(CAP))
        def _():
            def cnt_gbuf(t):
                tv = jnp.full((16,), t, jnp.int32)

                def cb(s, a):
                    def cu(u, aa):
                        v = gbuf_v[s, pl.ds(16 * u, 16)]
                        return aa + _popcnt(_key16(v) >= tv)
                    return lax.fori_loop(0, SEGW // 16, cu, a)
                return lax.fori_loop(0, n_pass, cb, jnp.int32(0))

            ttrue = _bsearch_kth(cnt_gbuf)
            ttrue_v = jnp.full((16,), ttrue, jnp.int32)
            lax.fori_loop(0, cval_v.shape[0] // 16, _clr_cand, 0)
            n_gt = append_pass(ttrue_v, True, False)

            def tb(s, ptr):
                gsid = plsc.load_gather(seglist_v,
                                        [jnp.full((16,), s, jnp.int32)])
                base = (gsid - rowseg) * SEGW

                def ub(u, pp):
                    v = gbuf_v[s, pl.ds(16 * u, 16)]
                    m = _key16(v) == ttrue_v
                    pe = jnp.minimum(pp, jnp.int32(CAP))
                    plsc.store_compressed(cval_v.at[pl.ds(pe, 16)], v, mask=m)
                    plsc.store_compressed(cidx_v.at[pl.ds(pe, 16)],
                                          base + (iota + 16 * u), mask=m)
                    return pp + _popcnt(m)
                return lax.fori_loop(0, SEGW // 16, ub, ptr)
            ptr_tie = lax.fori_loop(0, n_pass, tb, n_gt)
            nval_sm[0] = jnp.minimum(ptr_tie, jnp.int32(CAP))

        nvalid = nval_sm[0]

        @pl.when(nvalid > jnp.int32(192))
        def _():
            nv = (nvalid + 15) // 16

            def cnt_cand(t):
                tv = jnp.full((16,), t, jnp.int32)

                def cb(i, a):
                    return a + _popcnt(_key16(cval_v[pl.ds(16 * i, 16)]) >= tv)
                return lax.fori_loop(0, nv, cb, jnp.int32(0))

            tb_ = _bsearch_kth(cnt_cand)
            tbf = _inv_key16(jnp.full((16,), tb_, jnp.int32))

            def comp(i, p):
                v = cval_v[pl.ds(16 * i, 16)]
                x = cidx_v[pl.ds(16 * i, 16)]
                m = v >= tbf
                pe = jnp.minimum(p, jnp.int32(CAP))
                plsc.store_compressed(cval_v.at[pl.ds(pe, 16)], v, mask=m)
                plsc.store_compressed(cidx_v.at[pl.ds(pe, 16)], x, mask=m)
                return p + _popcnt(m)
            n2 = jnp.minimum(lax.fori_loop(0, nv, comp, jnp.int32(0)),
                             jnp.int32(CAP))

            def clr2(i, _):
                idx = n2 + 16 * i
                cval_v[pl.ds(idx, 16)] = ninf_v
                return 0
            lax.fori_loop(0, nv - (n2 // 16), clr2, 0)
            nval_sm[0] = n2

        nvalid = nval_sm[0]
        nv = (nvalid + 15) // 16

        def eb(k, _):
            def mb(i, mv):
                return jnp.maximum(mv, cval_v[pl.ds(16 * i, 16)])
            m = jnp.max(lax.fori_loop(0, nv, mb, ninf_v))
            msp = jnp.full((16,), m, jnp.float32)

            def ib(i, jv):
                v = cval_v[pl.ds(16 * i, 16)]
                x = cidx_v[pl.ds(16 * i, 16)]
                return jnp.minimum(jv, jnp.where(v == msp, x,
                                                 jnp.int32(IMAX)))
            j = jnp.min(lax.fori_loop(0, nv, ib,
                                      jnp.full((16,), IMAX, jnp.int32)))
            jsp = jnp.full((16,), j, jnp.int32)

            def ob(i, _):
                x = cidx_v[pl.ds(16 * i, 16)]
                v = cval_v[pl.ds(16 * i, 16)]
                cval_v[pl.ds(16 * i, 16)] = jnp.where(x == jsp, ninf_v, v)
                return 0
            lax.fori_loop(0, nv, ob, 0)
            plsc.store_scatter(outrow_v, [jnp.full((16,), k, jnp.int32)],
                               jsp, mask=lane0)
            return 0
        lax.fori_loop(0, K, eb, 0)

        pltpu.sync_copy(outrow_v, out_hbm.at[row])
        return 0

    lax.fori_loop(0, ROWS_PER_W, do_row, 0)


def _sc_topk(segs, segmax):
    mesh = plsc.VectorSubcoreMesh(core_axis_name="c", subcore_axis_name="s")
    return pl.kernel(
        _sc_topk_body,
        mesh=mesh,
        out_type=jax.ShapeDtypeStruct((B, 128), jnp.int32),
        scratch_types=[
            pltpu.VMEM((NSEG,), jnp.float32),
            pltpu.VMEM((NSEG,), jnp.int32),
            pltpu.VMEM((7 * 128 + 16,), jnp.int32),
            pltpu.VMEM((7 * 128, 128), jnp.float32),
            pltpu.VMEM(((CAP + 128),), jnp.float32),
            pltpu.VMEM(((CAP + 128),), jnp.int32),
            pltpu.VMEM((128,), jnp.int32),
            pltpu.SMEM((1,), jnp.int32),
            pltpu.SemaphoreType.DMA,
        ],
        compiler_params=pltpu.CompilerParams(use_tc_tiling_on_sc=False,
                                             needs_layout_passes=False),
    )(segs, segmax)


def kernel(user_id, user_features, user_history, user_id_table,
           W1, b1, W2, b2, Wt, bt, corpus):
    del user_history
    ue = _sc_gather(user_id_table, user_id)
    corpus_p = jnp.pad(corpus, ((0, CP - CORPUS), (0, 0)))
    scores, segmax3 = _tc_scores(ue, user_features, W1, b1[None, :], W2,
                                 b2[None, :], Wt[:DU], Wt[DU:], bt[None, :],
                                 corpus_p)
    segmax = segmax3.transpose(1, 0, 2).reshape(B, NSEG)
    segs = scores.reshape(B * NSEG, SEGW)
    out = _sc_topk(segs, segmax)
    return out[:, :K]


# speedup vs baseline: 11.4153x; 1.0025x over previous
import functools

import jax
import jax.numpy as jnp
from jax import lax
from jax.experimental import pallas as pl
from jax.experimental.pallas import tpu as pltpu
from jax.experimental.pallas import tpu_sc as plsc

B = 1024
IU = 128
DU = 32
DI = 32
HID = 256
CORPUS = 100000
K = 100

NC = 2
NS = 16
NW = NC * NS
ROWS_PER_W = B // NW

CB = 2048
CP = ((CORPUS + CB - 1) // CB) * CB
SEGW = 128
NSEG = CP // SEGW
NSEGV = NSEG // 16
CAP = 512
NEG = -3.0e38
NINF = float("-inf")
IMAX = 2147483647
IMIN = -2147483648


def _sc_gather_body(table_hbm, idx_hbm, out_hbm, idx_v, rows_v, sem):
    wid = lax.axis_index("s") * NC + lax.axis_index("c")
    base = wid * ROWS_PER_W
    pltpu.sync_copy(idx_hbm.at[pl.ds(base, ROWS_PER_W)], idx_v)
    pltpu.async_copy(table_hbm.at[idx_v], rows_v, sem).wait()
    pltpu.sync_copy(rows_v, out_hbm.at[pl.ds(base, ROWS_PER_W)])


def _sc_gather(table, idx):
    mesh = plsc.VectorSubcoreMesh(core_axis_name="c", subcore_axis_name="s")
    return pl.kernel(
        _sc_gather_body,
        mesh=mesh,
        out_type=jax.ShapeDtypeStruct((B, DU), jnp.float32),
        scratch_types=[
            pltpu.VMEM((ROWS_PER_W,), jnp.int32),
            pltpu.VMEM((ROWS_PER_W, DU), jnp.float32),
            pltpu.SemaphoreType.DMA,
        ],
        compiler_params=pltpu.CompilerParams(use_tc_tiling_on_sc=False),
    )(table, idx)


def _tc_score_body(ue_ref, uf_ref, w1_ref, b1_ref, w2_ref, b2_ref,
                   wta_ref, wtb_ref, bt_ref, corpus_ref, scores_ref,
                   segmax_ref, uemb_ref):
    i = pl.program_id(0)

    @pl.when(i == 0)
    def _():
        h = jnp.maximum(
            jnp.dot(uf_ref[...], w1_ref[...],
                    preferred_element_type=jnp.float32) + b1_ref[...], 0.0)
        ufe = jnp.dot(h, w2_ref[...],
                      preferred_element_type=jnp.float32) + b2_ref[...]
        uemb_ref[...] = (
            jnp.dot(ue_ref[...], wta_ref[...],
                    preferred_element_type=jnp.float32)
            + jnp.dot(ufe, wtb_ref[...], preferred_element_type=jnp.float32)
            + bt_ref[...])

    s = lax.dot_general(uemb_ref[...], corpus_ref[...],
                        (((1,), (1,)), ((), ())),
                        preferred_element_type=jnp.float32)
    col = i * CB + lax.broadcasted_iota(jnp.int32, (B, CB), 1)
    s = jnp.where(col < CORPUS, s, NEG)
    scores_ref[...] = s
    segmax_ref[...] = jnp.max(s.reshape(B, CB // SEGW, SEGW), axis=2)[None]


def _tc_scores(ue, uf, w1, b1, w2, b2, wta, wtb, bt, corpus_p):
    grid = CP // CB
    return pl.pallas_call(
        _tc_score_body,
        grid=(grid,),
        in_specs=[
            pl.BlockSpec((B, DU), lambda i: (0, 0)),
            pl.BlockSpec((B, IU), lambda i: (0, 0)),
            pl.BlockSpec((IU, HID), lambda i: (0, 0)),
            pl.BlockSpec((1, HID), lambda i: (0, 0)),
            pl.BlockSpec((HID, DU), lambda i: (0, 0)),
            pl.BlockSpec((1, DU), lambda i: (0, 0)),
            pl.BlockSpec((DU, DI), lambda i: (0, 0)),
            pl.BlockSpec((DU, DI), lambda i: (0, 0)),
            pl.BlockSpec((1, DI), lambda i: (0, 0)),
            pl.BlockSpec((CB, DI), lambda i: (i, 0)),
        ],
        out_specs=[
            pl.BlockSpec((B, CB), lambda i: (0, i)),
            pl.BlockSpec((1, B, CB // SEGW), lambda i: (i, 0, 0)),
        ],
        out_shape=[
            jax.ShapeDtypeStruct((B, CP), jnp.float32),
            jax.ShapeDtypeStruct((grid, B, CB // SEGW), jnp.float32),
        ],
        scratch_shapes=[pltpu.VMEM((B, DI), jnp.float32)],
    )(ue, uf, w1, b1, w2, b2, wta, wtb, bt, corpus_p)


def _key16(v):
    bits = plsc.bitcast(v, jnp.int32)
    return jnp.where(bits >= 0, bits, bits ^ jnp.int32(0x7FFFFFFF))


def _inv_key16(kv):
    return plsc.bitcast(
        jnp.where(kv >= 0, kv, kv ^ jnp.int32(0x7FFFFFFF)), jnp.float32)


def _bsearch_kth(count_ge):
    cpos = count_ge(jnp.int32(0))
    t0 = jnp.where(cpos >= K, jnp.int32(0), jnp.int32(IMIN))

    def bit_body(b, t):
        bit = lax.shift_left(jnp.int32(1), jnp.int32(30) - b)
        tc = t + bit
        return jnp.where(count_ge(tc) >= K, tc, t)

    return lax.fori_loop(0, 31, bit_body, t0)


def _popcnt(m):
    return plsc.all_reduce_population_count(m)[0]


def _sc_topk_body(segs_hbm, segmax_hbm, out_hbm,
                  smax_v, skey_v, seglist_v, gbuf_v, cval_v, cidx_v,
                  outrow_v, nval_sm, sem):
    wid = lax.axis_index("s") * NC + lax.axis_index("c")
    iota = lax.iota(jnp.int32, 16)
    lane0 = iota == jnp.int32(0)
    ninf_v = jnp.full((16,), NINF, jnp.float32)

    def _clr_seg(i, _):
        seglist_v[pl.ds(16 * i, 16)] = jnp.zeros((16,), jnp.int32)
        return 0
    lax.fori_loop(0, seglist_v.shape[0] // 16, _clr_seg, 0)

    def do_row(r, _):
        row = wid * ROWS_PER_W + r
        rowseg = row * NSEG

        pltpu.sync_copy(segmax_hbm.at[row], smax_v)

        def kb(i, _):
            skey_v[pl.ds(16 * i, 16)] = _key16(smax_v[pl.ds(16 * i, 16)])
            return 0
        lax.fori_loop(0, NSEGV, kb, 0)

        def cnt_skey(t):
            tv = jnp.full((16,), t, jnp.int32)

            def cb(i, a):
                return a + _popcnt(skey_v[pl.ds(16 * i, 16)] >= tv)
            return lax.fori_loop(0, NSEGV, cb, jnp.int32(0))

        tseg = _bsearch_kth(cnt_skey)
        tseg_v = jnp.full((16,), tseg, jnp.int32)
        t0f = _inv_key16(tseg_v)

        def sb(i, p):
            m = skey_v[pl.ds(16 * i, 16)] >= tseg_v
            plsc.store_compressed(seglist_v.at[pl.ds(p, 16)],
                                  iota + (16 * i + rowseg), mask=m)
            return p + _popcnt(m)
        n_pass = lax.fori_loop(0, NSEGV, sb, jnp.int32(0))

        def gb(c, _):
            pltpu.async_copy(
                segs_hbm.at[seglist_v.at[pl.ds(128 * c, 128)]],
                gbuf_v.at[pl.ds(128 * c, 128)], sem).wait()
            return 0
        lax.fori_loop(0, (n_pass + 127) // 128, gb, 0)

        def _clr_cand(i, _):
            cval_v[pl.ds(16 * i, 16)] = ninf_v
            return 0

        def append_pass(thresh_v, strict, equal):
            def ab(s, ptr):
                gsid = plsc.load_gather(seglist_v,
                                        [jnp.full((16,), s, jnp.int32)])
                base = (gsid - rowseg) * SEGW

                def ub(u, pp):
                    v = gbuf_v[s, pl.ds(16 * u, 16)]
                    if equal:
                        m = _key16(v) == thresh_v
                    elif strict:
                        m = _key16(v) > thresh_v
                    else:
                        m = v >= _inv_key16(thresh_v)
                    pe = jnp.minimum(pp, jnp.int32(CAP))
                    plsc.store_compressed(cval_v.at[pl.ds(pe, 16)], v, mask=m)
                    plsc.store_compressed(cidx_v.at[pl.ds(pe, 16)],
                                          base + (iota + 16 * u), mask=m)
                    return pp + _popcnt(m)
                return lax.fori_loop(0, SEGW // 16, ub, ptr)
            return lax.fori_loop(0, n_pass, ab, jnp.int32(0))

        lax.fori_loop(0, cval_v.shape[0] // 16, _clr_cand, 0)
        ptr_fast = append_pass(tseg_v, False, False)
        nval_sm[0] = jnp.minimum(ptr_fast, jnp.int32(CAP))

        @pl.when(ptr_fast > jnp.int32(CAP))
        def _():
            def cnt_gbuf(t):
                tv = jnp.full((16,), t, jnp.int32)

                def cb(s, a):
                    def cu(u, aa):
                        v = gbuf_v[s, pl.ds(16 * u, 16)]
                        return aa + _popcnt(_key16(v) >= tv)
                    return lax.fori_loop(0, SEGW // 16, cu, a)
                return lax.fori_loop(0, n_pass, cb, jnp.int32(0))

            ttrue = _bsearch_kth(cnt_gbuf)
            ttrue_v = jnp.full((16,), ttrue, jnp.int32)
            lax.fori_loop(0, cval_v.shape[0] // 16, _clr_cand, 0)
            n_gt = append_pass(ttrue_v, True, False)

            def tb(s, ptr):
                gsid = plsc.load_gather(seglist_v,
                                        [jnp.full((16,), s, jnp.int32)])
                base = (gsid - rowseg) * SEGW

                def ub(u, pp):
                    v = gbuf_v[s, pl.ds(16 * u, 16)]
                    m = _key16(v) == ttrue_v
                    pe = jnp.minimum(pp, jnp.int32(CAP))
                    plsc.store_compressed(cval_v.at[pl.ds(pe, 16)], v, mask=m)
                    plsc.store_compressed(cidx_v.at[pl.ds(pe, 16)],
                                          base + (iota + 16 * u), mask=m)
                    return pp + _popcnt(m)
                return lax.fori_loop(0, SEGW // 16, ub, ptr)
            ptr_tie = lax.fori_loop(0, n_pass, tb, n_gt)
            nval_sm[0] = jnp.minimum(ptr_tie, jnp.int32(CAP))

        nvalid = nval_sm[0]

        @pl.when(nvalid > jnp.int32(192))
        def _():
            nv = (nvalid + 15) // 16

            def cnt_cand(t):
                tv = jnp.full((16,), t, jnp.int32)

                def cb(i, a):
                    return a + _popcnt(_key16(cval_v[pl.ds(16 * i, 16)]) >= tv)
                return lax.fori_loop(0, nv, cb, jnp.int32(0))

            tb_ = _bsearch_kth(cnt_cand)
            tbf = _inv_key16(jnp.full((16,), tb_, jnp.int32))

            def comp(i, p):
                v = cval_v[pl.ds(16 * i, 16)]
                x = cidx_v[pl.ds(16 * i, 16)]
                m = v >= tbf
                pe = jnp.minimum(p, jnp.int32(CAP))
                plsc.store_compressed(cval_v.at[pl.ds(pe, 16)], v, mask=m)
                plsc.store_compressed(cidx_v.at[pl.ds(pe, 16)], x, mask=m)
                return p + _popcnt(m)
            n2 = jnp.minimum(lax.fori_loop(0, nv, comp, jnp.int32(0)),
                             jnp.int32(CAP))

            def clr2(i, _):
                idx = n2 + 16 * i
                cval_v[pl.ds(idx, 16)] = ninf_v
                return 0
            lax.fori_loop(0, nv - (n2 // 16), clr2, 0)
            nval_sm[0] = n2

        nvalid = nval_sm[0]
        nv = (nvalid + 15) // 16

        def eb(k, _):
            def mb(i, mv):
                return jnp.maximum(mv, cval_v[pl.ds(16 * i, 16)])
            m = jnp.max(lax.fori_loop(0, nv, mb, ninf_v))
            msp = jnp.full((16,), m, jnp.float32)

            def ib(i, jv):
                v = cval_v[pl.ds(16 * i, 16)]
                x = cidx_v[pl.ds(16 * i, 16)]
                return jnp.minimum(jv, jnp.where(v == msp, x,
                                                 jnp.int32(IMAX)))
            j = jnp.min(lax.fori_loop(0, nv, ib,
                                      jnp.full((16,), IMAX, jnp.int32)))
            jsp = jnp.full((16,), j, jnp.int32)

            def ob(i, _):
                x = cidx_v[pl.ds(16 * i, 16)]
                v = cval_v[pl.ds(16 * i, 16)]
                cval_v[pl.ds(16 * i, 16)] = jnp.where(x == jsp, ninf_v, v)
                return 0
            lax.fori_loop(0, nv, ob, 0)
            plsc.store_scatter(outrow_v, [jnp.full((16,), k, jnp.int32)],
                               jsp, mask=lane0)
            return 0
        lax.fori_loop(0, K, eb, 0)

        pltpu.sync_copy(outrow_v, out_hbm.at[row])
        return 0

    lax.fori_loop(0, ROWS_PER_W, do_row, 0)


def _sc_topk(segs, segmax):
    mesh = plsc.VectorSubcoreMesh(core_axis_name="c", subcore_axis_name="s")
    return pl.kernel(
        _sc_topk_body,
        mesh=mesh,
        out_type=jax.ShapeDtypeStruct((B, 128), jnp.int32),
        scratch_types=[
            pltpu.VMEM((NSEG,), jnp.float32),
            pltpu.VMEM((NSEG,), jnp.int32),
            pltpu.VMEM((7 * 128 + 16,), jnp.int32),
            pltpu.VMEM((7 * 128, 128), jnp.float32),
            pltpu.VMEM(((CAP + 128),), jnp.float32),
            pltpu.VMEM(((CAP + 128),), jnp.int32),
            pltpu.VMEM((128,), jnp.int32),
            pltpu.SMEM((1,), jnp.int32),
            pltpu.SemaphoreType.DMA,
        ],
        compiler_params=pltpu.CompilerParams(use_tc_tiling_on_sc=True,
                                             needs_layout_passes=False),
    )(segs, segmax)


def kernel(user_id, user_features, user_history, user_id_table,
           W1, b1, W2, b2, Wt, bt, corpus):
    del user_history
    ue = _sc_gather(user_id_table, user_id)
    corpus_p = jnp.pad(corpus, ((0, CP - CORPUS), (0, 0)))
    scores, segmax3 = _tc_scores(ue, user_features, W1, b1[None, :], W2,
                                 b2[None, :], Wt[:DU], Wt[DU:], bt[None, :],
                                 corpus_p)
    segmax = segmax3.transpose(1, 0, 2).reshape(B, NSEG)
    segs = scores.reshape(B * NSEG, SEGW)
    out = _sc_topk(segs, segmax)
    return out[:, :K]


# speedup vs baseline: 11.6343x; 1.0192x over previous
import functools

import jax
import jax.numpy as jnp
from jax import lax
from jax.experimental import pallas as pl
from jax.experimental.pallas import tpu as pltpu
from jax.experimental.pallas import tpu_sc as plsc

B = 1024
IU = 128
DU = 32
DI = 32
HID = 256
CORPUS = 100000
K = 100

NC = 2
NS = 16
NW = NC * NS
ROWS_PER_W = B // NW

CB = 2048
CP = ((CORPUS + CB - 1) // CB) * CB
SEGW = 128
NSEG = CP // SEGW
NSEGV = NSEG // 16
CAP = 512
NEG = -3.0e38
NINF = float("-inf")
IMAX = 2147483647
IMIN = -2147483648


def _sc_gather_body(table_hbm, idx_hbm, out_hbm, idx_v, rows_v, sem):
    wid = lax.axis_index("s") * NC + lax.axis_index("c")
    base = wid * ROWS_PER_W
    pltpu.sync_copy(idx_hbm.at[pl.ds(base, ROWS_PER_W)], idx_v)
    pltpu.async_copy(table_hbm.at[idx_v], rows_v, sem).wait()
    pltpu.sync_copy(rows_v, out_hbm.at[pl.ds(base, ROWS_PER_W)])


def _sc_gather(table, idx):
    mesh = plsc.VectorSubcoreMesh(core_axis_name="c", subcore_axis_name="s")
    return pl.kernel(
        _sc_gather_body,
        mesh=mesh,
        out_type=jax.ShapeDtypeStruct((B, DU), jnp.float32),
        scratch_types=[
            pltpu.VMEM((ROWS_PER_W,), jnp.int32),
            pltpu.VMEM((ROWS_PER_W, DU), jnp.float32),
            pltpu.SemaphoreType.DMA,
        ],
        compiler_params=pltpu.CompilerParams(use_tc_tiling_on_sc=False),
    )(table, idx)


def _tc_score_body(ue_ref, uf_ref, w1_ref, b1_ref, w2_ref, b2_ref,
                   wta_ref, wtb_ref, bt_ref, corpus_ref, scores_ref,
                   segmax_ref, uemb_ref):
    i = pl.program_id(0)

    @pl.when(i == 0)
    def _():
        h = jnp.maximum(
            jnp.dot(uf_ref[...], w1_ref[...],
                    preferred_element_type=jnp.float32) + b1_ref[...], 0.0)
        ufe = jnp.dot(h, w2_ref[...],
                      preferred_element_type=jnp.float32) + b2_ref[...]
        uemb_ref[...] = (
            jnp.dot(ue_ref[...], wta_ref[...],
                    preferred_element_type=jnp.float32)
            + jnp.dot(ufe, wtb_ref[...], preferred_element_type=jnp.float32)
            + bt_ref[...])

    s = lax.dot_general(uemb_ref[...], corpus_ref[...],
                        (((1,), (1,)), ((), ())),
                        preferred_element_type=jnp.float32)
    col = i * CB + lax.broadcasted_iota(jnp.int32, (B, CB), 1)
    s = jnp.where(col < CORPUS, s, NEG)
    scores_ref[...] = s
    segmax_ref[...] = jnp.max(s.reshape(B, CB // SEGW, SEGW), axis=2)[None]


def _tc_scores(ue, uf, w1, b1, w2, b2, wta, wtb, bt, corpus_p):
    grid = CP // CB
    return pl.pallas_call(
        _tc_score_body,
        grid=(grid,),
        in_specs=[
            pl.BlockSpec((B, DU), lambda i: (0, 0)),
            pl.BlockSpec((B, IU), lambda i: (0, 0)),
            pl.BlockSpec((IU, HID), lambda i: (0, 0)),
            pl.BlockSpec((1, HID), lambda i: (0, 0)),
            pl.BlockSpec((HID, DU), lambda i: (0, 0)),
            pl.BlockSpec((1, DU), lambda i: (0, 0)),
            pl.BlockSpec((DU, DI), lambda i: (0, 0)),
            pl.BlockSpec((DU, DI), lambda i: (0, 0)),
            pl.BlockSpec((1, DI), lambda i: (0, 0)),
            pl.BlockSpec((CB, DI), lambda i: (i, 0)),
        ],
        out_specs=[
            pl.BlockSpec((B, CB), lambda i: (0, i)),
            pl.BlockSpec((1, B, CB // SEGW), lambda i: (i, 0, 0)),
        ],
        out_shape=[
            jax.ShapeDtypeStruct((B, CP), jnp.float32),
            jax.ShapeDtypeStruct((grid, B, CB // SEGW), jnp.float32),
        ],
        scratch_shapes=[pltpu.VMEM((B, DI), jnp.float32)],
    )(ue, uf, w1, b1, w2, b2, wta, wtb, bt, corpus_p)


def _key16(v):
    bits = plsc.bitcast(v, jnp.int32)
    return jnp.where(bits >= 0, bits, bits ^ jnp.int32(0x7FFFFFFF))


def _inv_key16(kv):
    return plsc.bitcast(
        jnp.where(kv >= 0, kv, kv ^ jnp.int32(0x7FFFFFFF)), jnp.float32)


def _bsearch_kth(count_ge):
    cpos = count_ge(jnp.int32(0))
    t0 = jnp.where(cpos >= K, jnp.int32(0), jnp.int32(IMIN))

    def bit_body(b, t):
        bit = lax.shift_left(jnp.int32(1), jnp.int32(30) - b)
        tc = t + bit
        return jnp.where(count_ge(tc) >= K, tc, t)

    return lax.fori_loop(0, 31, bit_body, t0)


def _popcnt(m):
    return plsc.all_reduce_population_count(m)[0]


NV_SMALL = 12


def _sc_topk_body(segs_hbm, segmax_hbm, out_hbm,
                  smax_v, skey_v, seglist_v, gbuf_v, cval_v, cidx_v,
                  outrow_v, nval_sm, sem):
    wid = lax.axis_index("s") * NC + lax.axis_index("c")
    iota = lax.iota(jnp.int32, 16)
    lane0 = iota == jnp.int32(0)
    ninf_v = jnp.full((16,), NINF, jnp.float32)
    one_v = jnp.full((16,), 1, jnp.int32)
    zero_v = jnp.zeros((16,), jnp.int32)

    for i in range(seglist_v.shape[0] // 16):
        seglist_v[pl.ds(16 * i, 16)] = zero_v

    def do_row(r, _):
        row = wid * ROWS_PER_W + r
        rowseg = row * NSEG

        pltpu.sync_copy(segmax_hbm.at[row], smax_v)

        for i in range(NSEGV):
            skey_v[pl.ds(16 * i, 16)] = _key16(smax_v[pl.ds(16 * i, 16)])

        def cnt_skey(t):
            tv = jnp.full((16,), t, jnp.int32)
            acc = zero_v
            for i in range(NSEGV):
                acc = acc + jnp.where(skey_v[pl.ds(16 * i, 16)] >= tv,
                                      one_v, zero_v)
            return jnp.sum(acc)

        tseg = _bsearch_kth(cnt_skey)
        tseg_v = jnp.full((16,), tseg, jnp.int32)
        t0f = _inv_key16(tseg_v)

        GRP = 10
        p = jnp.int32(0)
        for g in range(0, NSEGV, GRP):
            n = min(GRP, NSEGV - g)
            ms = [skey_v[pl.ds(16 * (g + j), 16)] >= tseg_v
                  for j in range(n)]
            cs = [_popcnt(m) for m in ms]
            offs = [p]
            for j in range(n - 1):
                offs.append(offs[-1] + cs[j])
            for j in range(n):
                plsc.store_compressed(seglist_v.at[pl.ds(offs[j], 16)],
                                      iota + (16 * (g + j) + rowseg),
                                      mask=ms[j])
            p = offs[-1] + cs[n - 1]
        n_pass = p

        def gb(c, _):
            pltpu.async_copy(
                segs_hbm.at[seglist_v.at[pl.ds(128 * c, 128)]],
                gbuf_v.at[pl.ds(128 * c, 128)], sem).wait()
            return 0
        lax.fori_loop(0, (n_pass + 127) // 128, gb, 0)

        def clr_cand():
            for i in range(34):
                cval_v[pl.ds(16 * i, 16)] = ninf_v

        def append_pass(thresh_v, mode):
            def ab(s, ptr):
                gsid = plsc.load_gather(seglist_v,
                                        [jnp.full((16,), s, jnp.int32)])
                base = (gsid - rowseg) * SEGW
                vs = [gbuf_v[s, pl.ds(16 * u, 16)] for u in range(8)]
                if mode == 0:
                    ms = [v >= t0f for v in vs]
                elif mode == 1:
                    ms = [_key16(v) > thresh_v for v in vs]
                else:
                    ms = [_key16(v) == thresh_v for v in vs]
                cs = [_popcnt(m) for m in ms]
                offs = [ptr]
                for u in range(7):
                    offs.append(offs[-1] + cs[u])
                for u in range(8):
                    pe = jnp.minimum(offs[u], jnp.int32(CAP))
                    plsc.store_compressed(cval_v.at[pl.ds(pe, 16)], vs[u],
                                          mask=ms[u])
                    plsc.store_compressed(cidx_v.at[pl.ds(pe, 16)],
                                          base + (iota + 16 * u), mask=ms[u])
                return offs[-1] + cs[7]
            return lax.fori_loop(0, n_pass, ab, jnp.int32(0))

        clr_cand()
        ptr_fast = append_pass(tseg_v, 0)
        nval_sm[0] = jnp.minimum(ptr_fast, jnp.int32(CAP))

        @pl.when(ptr_fast > jnp.int32(CAP))
        def _():
            def cnt_gbuf(t):
                tv = jnp.full((16,), t, jnp.int32)

                def cb(s, a):
                    acc = zero_v
                    for u in range(8):
                        v = gbuf_v[s, pl.ds(16 * u, 16)]
                        acc = acc + jnp.where(_key16(v) >= tv, one_v, zero_v)
                    return a + jnp.sum(acc)
                return lax.fori_loop(0, n_pass, cb, jnp.int32(0))

            ttrue = _bsearch_kth(cnt_gbuf)
            ttrue_v = jnp.full((16,), ttrue, jnp.int32)
            clr_cand()
            n_gt = append_pass(ttrue_v, 1)
            def tie(s, ptr):
                gsid = plsc.load_gather(seglist_v,
                                        [jnp.full((16,), s, jnp.int32)])
                base = (gsid - rowseg) * SEGW
                pp = ptr
                for u in range(8):
                    v = gbuf_v[s, pl.ds(16 * u, 16)]
                    m = _key16(v) == ttrue_v
                    pe = jnp.minimum(pp, jnp.int32(CAP))
                    plsc.store_compressed(cval_v.at[pl.ds(pe, 16)], v, mask=m)
                    plsc.store_compressed(cidx_v.at[pl.ds(pe, 16)],
                                          base + (iota + 16 * u), mask=m)
                    pp = pp + _popcnt(m)
                return pp
            ptr_tie = lax.fori_loop(0, n_pass, tie, n_gt)
            nval_sm[0] = jnp.minimum(ptr_tie, jnp.int32(CAP))

        nvalid = nval_sm[0]

        @pl.when(nvalid > jnp.int32(16 * NV_SMALL))
        def _():
            def cnt_cand(t):
                tv = jnp.full((16,), t, jnp.int32)
                acc = zero_v
                for i in range(CAP // 16):
                    acc = acc + jnp.where(
                        _key16(cval_v[pl.ds(16 * i, 16)]) >= tv,
                        one_v, zero_v)
                return jnp.sum(acc)

            tb_ = _bsearch_kth(cnt_cand)
            tbf = _inv_key16(jnp.full((16,), tb_, jnp.int32))

            p2 = jnp.int32(0)
            for i in range(CAP // 16):
                v = cval_v[pl.ds(16 * i, 16)]
                x = cidx_v[pl.ds(16 * i, 16)]
                m = v >= tbf
                pe = jnp.minimum(p2, jnp.int32(CAP))
                plsc.store_compressed(cval_v.at[pl.ds(pe, 16)], v, mask=m)
                plsc.store_compressed(cidx_v.at[pl.ds(pe, 16)], x, mask=m)
                p2 = p2 + _popcnt(m)
            n2 = jnp.minimum(p2, jnp.int32(CAP))
            for i in range(26):
                cval_v[pl.ds(n2 + 16 * i, 16)] = ninf_v
            nval_sm[0] = n2

        nvalid = nval_sm[0]

        def extract(nvreg):
            def eb(k, _):
                mv = ninf_v
                for i in range(nvreg):
                    mv = jnp.maximum(mv, cval_v[pl.ds(16 * i, 16)])
                msp = jnp.full((16,), jnp.max(mv), jnp.float32)
                jv = jnp.full((16,), IMAX, jnp.int32)
                for i in range(nvreg):
                    v = cval_v[pl.ds(16 * i, 16)]
                    x = cidx_v[pl.ds(16 * i, 16)]
                    jv = jnp.minimum(jv, jnp.where(v == msp, x,
                                                   jnp.int32(IMAX)))
                jsp = jnp.full((16,), jnp.min(jv), jnp.int32)
                for i in range(nvreg):
                    x = cidx_v[pl.ds(16 * i, 16)]
                    v = cval_v[pl.ds(16 * i, 16)]
                    cval_v[pl.ds(16 * i, 16)] = jnp.where(x == jsp, ninf_v, v)
                plsc.store_scatter(outrow_v, [jnp.full((16,), k, jnp.int32)],
                                   jsp, mask=lane0)
                return 0
            lax.fori_loop(0, K, eb, 0)

        @pl.when(nvalid <= jnp.int32(16 * NV_SMALL))
        def _():
            extract(NV_SMALL)

        @pl.when(nvalid > jnp.int32(16 * NV_SMALL))
        def _():
            extract(CAP // 16)

        pltpu.sync_copy(outrow_v, out_hbm.at[row])
        return 0

    lax.fori_loop(0, ROWS_PER_W, do_row, 0)


def _sc_topk(segs, segmax):
    mesh = plsc.VectorSubcoreMesh(core_axis_name="c", subcore_axis_name="s")
    return pl.kernel(
        _sc_topk_body,
        mesh=mesh,
        out_type=jax.ShapeDtypeStruct((B, 128), jnp.int32),
        scratch_types=[
            pltpu.VMEM((NSEG,), jnp.float32),
            pltpu.VMEM((NSEG,), jnp.int32),
            pltpu.VMEM((7 * 128 + 16,), jnp.int32),
            pltpu.VMEM((7 * 128, 128), jnp.float32),
            pltpu.VMEM((1040,), jnp.float32),
            pltpu.VMEM((1040,), jnp.int32),
            pltpu.VMEM((128,), jnp.int32),
            pltpu.SMEM((1,), jnp.int32),
            pltpu.SemaphoreType.DMA,
        ],
        compiler_params=pltpu.CompilerParams(use_tc_tiling_on_sc=True,
                                             needs_layout_passes=False),
    )(segs, segmax)


def kernel(user_id, user_features, user_history, user_id_table,
           W1, b1, W2, b2, Wt, bt, corpus):
    del user_history
    ue = _sc_gather(user_id_table, user_id)
    corpus_p = jnp.pad(corpus, ((0, CP - CORPUS), (0, 0)))
    scores, segmax3 = _tc_scores(ue, user_features, W1, b1[None, :], W2,
                                 b2[None, :], Wt[:DU], Wt[DU:], bt[None, :],
                                 corpus_p)
    segmax = segmax3.transpose(1, 0, 2).reshape(B, NSEG)
    segs = scores.reshape(B * NSEG, SEGW)
    out = _sc_topk(segs, segmax)
    return out[:, :K]
